# baseline (speedup 1.0000x reference)
import functools

import jax
import jax.numpy as jnp
from jax import lax
from jax.experimental import pallas as pl
from jax.experimental.pallas import tpu as pltpu
from jax.experimental.pallas import tpu_sc as plsc

VOCAB = 1000000
D = 64
BATCH = 4096
HIST = 200
B = BATCH * HIST

_info = plsc.get_sparse_core_info()
NC = _info.num_cores
NS = _info.num_subcores
NW = NC * NS
BPW = B // NW
CHUNK = 512
NCHUNK = BPW // CHUNK

_mesh = plsc.VectorSubcoreMesh(core_axis_name="c", subcore_axis_name="s")


@functools.partial(
    pl.kernel,
    mesh=_mesh,
    out_type=jax.ShapeDtypeStruct((B, D), jnp.float32),
    scratch_types=[
        pltpu.VMEM((BPW,), jnp.int32),
        pltpu.VMEM((CHUNK, D), jnp.float32),
        pltpu.SemaphoreType.DMA,
    ],
    compiler_params=pltpu.CompilerParams(use_tc_tiling_on_sc=False),
)
def _gather_kernel(table_hbm, idx_hbm, out_hbm, idx_v, rows_v, sem):
    wid = lax.axis_index("s") * NC + lax.axis_index("c")
    base = wid * BPW
    pltpu.sync_copy(idx_hbm.at[pl.ds(base, BPW)], idx_v)

    def body(i, carry):
        off = i * CHUNK
        pltpu.async_copy(
            table_hbm.at[idx_v.at[pl.ds(off, CHUNK)]], rows_v, sem
        ).wait()
        pltpu.sync_copy(rows_v, out_hbm.at[pl.ds(base + off, CHUNK)])
        return carry

    lax.fori_loop(0, NCHUNK, body, 0)


def kernel(x, table):
    flat_idx = x.reshape(B).astype(jnp.int32)
    out = _gather_kernel(table, flat_idx)
    return out.reshape(BATCH, HIST, D)


# speedup vs baseline: 1.0244x; 1.0244x over previous
import functools

import jax
import jax.numpy as jnp
from jax import lax
from jax.experimental import pallas as pl
from jax.experimental.pallas import tpu as pltpu
from jax.experimental.pallas import tpu_sc as plsc

VOCAB = 1000000
D = 64
BATCH = 4096
HIST = 200
B = BATCH * HIST

_info = plsc.get_sparse_core_info()
NC = _info.num_cores
NS = _info.num_subcores
NW = NC * NS
BPW = B // NW

C = 256
K = 2
GC = K * C
NG = BPW // GC
NT = NG // 2

_mesh = plsc.VectorSubcoreMesh(core_axis_name="c", subcore_axis_name="s")


@functools.partial(
    pl.kernel,
    mesh=_mesh,
    out_type=jax.ShapeDtypeStruct((B, D), jnp.float32),
    scratch_types=[
        pltpu.VMEM((BPW,), jnp.int32),
        pltpu.VMEM((2 * GC, D), jnp.float32),
        pltpu.SemaphoreType.DMA,
        pltpu.SemaphoreType.DMA,
    ],
    compiler_params=pltpu.CompilerParams(use_tc_tiling_on_sc=False),
)
def _gather_kernel(table_hbm, idx_hbm, out_hbm, idx_v, rows_v, sem_g, sem_o):
    wid = lax.axis_index("s") * NC + lax.axis_index("c")
    base = wid * BPW
    pltpu.sync_copy(idx_hbm.at[pl.ds(base, BPW)], idx_v)

    def fire_gathers(g, half):
        for j in range(K):
            pltpu.async_copy(
                table_hbm.at[idx_v.at[pl.ds(g * GC + j * C, C)]],
                rows_v.at[pl.ds(half * GC + j * C, C)],
                sem_g,
            )

    def fire_outs(g, half):
        for j in range(K):
            pltpu.async_copy(
                rows_v.at[pl.ds(half * GC + j * C, C)],
                out_hbm.at[pl.ds(base + g * GC + j * C, C)],
                sem_o,
            )

    def drain(sem, n):
        for _ in range(n):
            pltpu.make_async_copy(
                out_hbm.at[pl.ds(0, C)], rows_v.at[pl.ds(0, C)], sem
            ).wait()

    fire_gathers(0, 0)

    def pair_body(t, carry):
        g0 = 2 * t
        drain(sem_g, K)
        fire_outs(g0, 0)
        pl.when(t > 0)(lambda: drain(sem_o, K))
        fire_gathers(g0 + 1, 1)
        drain(sem_g, K)
        fire_outs(g0 + 1, 1)

        def refill():
            drain(sem_o, K)
            fire_gathers(g0 + 2, 0)

        pl.when(t < NT - 1)(refill)
        return carry

    lax.fori_loop(0, NT, pair_body, 0)
    drain(sem_o, 2 * K)


def kernel(x, table):
    flat_idx = x.reshape(B).astype(jnp.int32)
    out = _gather_kernel(table, flat_idx)
    return out.reshape(BATCH, HIST, D)


# speedup vs baseline: 1.3641x; 1.3316x over previous
import functools

import jax
import jax.numpy as jnp
from jax import lax
from jax.experimental import pallas as pl
from jax.experimental.pallas import tpu as pltpu
from jax.experimental.pallas import tpu_sc as plsc

VOCAB = 1000000
D = 64
BATCH = 4096
HIST = 200
B = BATCH * HIST

_info = plsc.get_sparse_core_info()
NC = _info.num_cores
NS = _info.num_subcores
NW = NC * NS
BPW = B // NW

C = 256
K = 2
GC = K * C
NG = BPW // GC
NT = NG // 2

_mesh = plsc.VectorSubcoreMesh(core_axis_name="c", subcore_axis_name="s")


@functools.partial(
    pl.kernel,
    mesh=_mesh,
    out_type=jax.ShapeDtypeStruct((B, 2 * D), jnp.float32),
    scratch_types=[
        pltpu.VMEM((BPW,), jnp.int32),
        pltpu.VMEM((2 * GC, D), jnp.float32),
        pltpu.SemaphoreType.DMA,
        pltpu.SemaphoreType.DMA,
    ],
    compiler_params=pltpu.CompilerParams(use_tc_tiling_on_sc=False),
)
def _gather_kernel(table_hbm, idx_hbm, out_hbm, idx_v, rows_v, sem_g, sem_o):
    wid = lax.axis_index("s") * NC + lax.axis_index("c")
    base = wid * BPW
    pltpu.sync_copy(idx_hbm.at[pl.ds(base, BPW)], idx_v)

    def fire_gathers(g, half):
        for j in range(K):
            pltpu.async_copy(
                table_hbm.at[idx_v.at[pl.ds(g * GC + j * C, C)]],
                rows_v.at[pl.ds(half * GC + j * C, C)],
                sem_g,
            )

    def fire_outs(g, half):
        for j in range(K):
            pltpu.async_copy(
                rows_v.at[pl.ds(half * GC + j * C, C)],
                out_hbm.at[pl.ds(base + g * GC + j * C, C), pl.ds(0, D)],
                sem_o,
            )

    def drain(sem, n):
        for _ in range(n):
            pltpu.make_async_copy(
                out_hbm.at[pl.ds(0, C), pl.ds(0, D)],
                rows_v.at[pl.ds(0, C)],
                sem,
            ).wait()

    fire_gathers(0, 0)

    def pair_body(t, carry):
        g0 = 2 * t
        drain(sem_g, K)
        fire_outs(g0, 0)
        pl.when(t > 0)(lambda: drain(sem_o, K))
        fire_gathers(g0 + 1, 1)
        drain(sem_g, K)
        fire_outs(g0 + 1, 1)

        def refill():
            drain(sem_o, K)
            fire_gathers(g0 + 2, 0)

        pl.when(t < NT - 1)(refill)
        return carry

    lax.fori_loop(0, NT, pair_body, 0)
    drain(sem_o, 2 * K)


def kernel(x, table):
    flat_idx = x.reshape(B).astype(jnp.int32)
    out = _gather_kernel(table, flat_idx)
    return out[:, :D].reshape(BATCH, HIST, D)


# speedup vs baseline: 1.3648x; 1.0006x over previous
import functools

import jax
import jax.numpy as jnp
from jax import lax
from jax.experimental import pallas as pl
from jax.experimental.pallas import tpu as pltpu
from jax.experimental.pallas import tpu_sc as plsc

VOCAB = 1000000
D = 64
BATCH = 4096
HIST = 200
B = BATCH * HIST

_info = plsc.get_sparse_core_info()
NC = _info.num_cores
NS = _info.num_subcores
NW = NC * NS
BPW = B // NW

C = 256
K = 2
GC = K * C
NG = BPW // GC
NT = NG // 2

_mesh = plsc.VectorSubcoreMesh(core_axis_name="c", subcore_axis_name="s")


@functools.partial(
    pl.kernel,
    mesh=_mesh,
    out_type=jax.ShapeDtypeStruct((B, 2 * D), jnp.float32),
    scratch_types=[
        pltpu.VMEM((BPW,), jnp.int32),
        pltpu.VMEM((2 * GC, D), jnp.float32),
        pltpu.SemaphoreType.DMA,
        pltpu.SemaphoreType.DMA,
    ],
    compiler_params=pltpu.CompilerParams(use_tc_tiling_on_sc=False),
)
def _gather_kernel(table_hbm, idx_hbm, out_hbm, idx_v, rows_v, sem_g, sem_o):
    wid = lax.axis_index("s") * NC + lax.axis_index("c")
    base = wid * BPW
    pltpu.sync_copy(idx_hbm.at[pl.ds(base, BPW)], idx_v)

    def fire_gathers(g, half):
        for j in range(K):
            pltpu.async_copy(
                table_hbm.at[idx_v.at[pl.ds(g * GC + j * C, C)]],
                rows_v.at[pl.ds(half * GC + j * C, C)],
                sem_g,
            )

    def fire_outs(g, half):
        for j in range(K):
            pltpu.async_copy(
                rows_v.at[pl.ds(half * GC + j * C, C)],
                out_hbm.at[pl.ds(base + g * GC + j * C, C), pl.ds(0, D)],
                sem_o,
            )

    def drain(sem, n):
        for _ in range(n):
            pltpu.make_async_copy(
                out_hbm.at[pl.ds(0, C), pl.ds(0, D)],
                rows_v.at[pl.ds(0, C)],
                sem,
            ).wait()

    fire_gathers(0, 0)

    def pair_body(t, carry):
        g0 = 2 * t
        drain(sem_g, K)
        fire_outs(g0, 0)
        pl.when(t > 0)(lambda: drain(sem_o, K))
        fire_gathers(g0 + 1, 1)
        drain(sem_g, K)
        fire_outs(g0 + 1, 1)

        def refill():
            drain(sem_o, K)
            fire_gathers(g0 + 2, 0)

        pl.when(t < NT - 1)(refill)
        return carry

    lax.fori_loop(0, NT, pair_body, 0)
    drain(sem_o, 2 * K)


def kernel(x, table):
    flat_idx = x.reshape(B).astype(jnp.int32)
    out = _gather_kernel(table, flat_idx)
    return out.reshape(BATCH, HIST, 2 * D)[..., :D]
